# baseline (speedup 1.0000x reference)
import jax
import jax.numpy as jnp
from jax import lax
from jax.experimental import pallas as pl
from jax.experimental.pallas import tpu as pltpu
from jax.experimental.pallas import tpu_sc as plsc

N_NODES_K = 100000
N_EDGES_K = 6400000
H_K = 4
CHUNK = 2048
SUBSC = CHUNK // 128
NCORES = 2
NSUB = 16
NW = NCORES * NSUB
NCH_PAD = 3136
NCHUNKS = N_EDGES_K // CHUNK
CPW = NCH_PAD // NW
NPS = 6272
N_PAD = NPS * NSUB
NBLK = 1568


def _bcast(v):
    return jnp.full((16,), v, jnp.float32)


def _vlrelu(v, a):
    return jnp.maximum(v, a * v)


def _sweep_loop(src3_hbm, dst3_hbm, bufs, x_sp, sz_sp, cs, cd, pp, wid):
    iota = lax.iota(jnp.int32, 16)
    lo = wid * CPW

    def fire_idx(ch, p):
        src2, dst2, isem = bufs[p][0], bufs[p][1], bufs[p][6]
        chc = jnp.minimum(ch, NCH_PAD - 1)
        pltpu.async_copy(src3_hbm.at[chc], src2, isem)
        pltpu.async_copy(dst3_hbm.at[chc], dst2, isem)

    def wait_idx(p):
        src2, dst2, isem = bufs[p][0], bufs[p][1], bufs[p][6]
        pltpu.make_async_copy(src3_hbm.at[0], src2, isem).wait()
        pltpu.make_async_copy(dst3_hbm.at[0], dst2, isem).wait()

    def fire_gathers(p):
        src2, dst2, xs_v, xd_v, gsem = (bufs[p][0], bufs[p][1], bufs[p][2],
                                        bufs[p][3], bufs[p][7])
        for j in range(SUBSC):
            pltpu.async_copy(x_sp.at[src2.at[j]], xs_v.at[j], gsem)
            pltpu.async_copy(x_sp.at[dst2.at[j]], xd_v.at[j], gsem)

    def wait_gathers(p):
        src2, dst2, xs_v, xd_v, gsem = (bufs[p][0], bufs[p][1], bufs[p][2],
                                        bufs[p][3], bufs[p][7])
        for j in range(SUBSC):
            pltpu.make_async_copy(x_sp.at[src2.at[j]], xs_v.at[j], gsem).wait()
            pltpu.make_async_copy(x_sp.at[dst2.at[j]], xd_v.at[j], gsem).wait()

    def fire_scatters(p):
        ds_v, out_v, ssem = bufs[p][4], bufs[p][5], bufs[p][8]
        for j in range(SUBSC):
            pltpu.async_copy(out_v.at[pl.ds(j * 128, 128)],
                             sz_sp.at[ds_v.at[j]], ssem, add=True)

    def drain_scatters(p):
        ds_v, out_v, ssem = bufs[p][4], bufs[p][5], bufs[p][8]
        for j in range(SUBSC):
            pltpu.make_async_copy(out_v.at[pl.ds(j * 128, 128)],
                                  sz_sp.at[ds_v.at[j]], ssem).wait()

    def compute(p):
        dst2, xs_v, xd_v, ds_v, out_v = (bufs[p][1], bufs[p][2], bufs[p][3],
                                         bufs[p][4], bufs[p][5])
        for j in range(SUBSC):
            @plsc.parallel_loop(0, 8, unroll=2)
            def grp(l):
                g = j * 8 + l
                xs = xs_v[j, pl.ds(l * 16, 16)]
                xd = xd_v[j, pl.ds(l * 16, 16)]
                ds_v[j, pl.ds(l * 16, 16)] = dst2[j, pl.ds(l * 16, 16)]
                row = g * 16 + iota
                for h in range(H_K):
                    cdxd = cd[h] * xd
                    u = cs[h] * xs + cdxd
                    a = jnp.maximum(u, 0.2 * u)
                    t = pp[h] + cdxd
                    b = jnp.maximum(t, 0.2 * t)
                    w = jnp.exp(a - b)
                    plsc.store_scatter(out_v,
                                       [row, jnp.full((16,), h, jnp.int32)], w)
                    plsc.store_scatter(out_v,
                                       [row, jnp.full((16,), H_K + h, jnp.int32)],
                                       xs * w)

    fire_idx(lo, 0)
    fire_idx(lo + 1, 1)

    def pair_body(it, carry):
        c0 = lo + 2 * it
        for p in range(2):
            wait_idx(p)
            fire_gathers(p)

            @pl.when(it > 0)
            def _():
                drain_scatters(p)

            wait_gathers(p)
            compute(p)
            fire_idx(c0 + p + 2, p)
            fire_scatters(p)
        return carry

    lax.fori_loop(0, CPW // 2, pair_body, 0)
    for p in range(2):
        drain_scatters(p)
        wait_idx(p)


def _writeback(out_hbm, sz_sp, c, s):
    plsc.subcore_barrier()
    pltpu.sync_copy(sz_sp.at[pl.ds(s * NPS, NPS)],
                    out_hbm.at[c, pl.ds(s * NPS, NPS)])


def _sweep0_body(x_hbm, src3_hbm, dst3_hbm, par_hbm, zeros_hbm, out_hbm,
                 srcA, dstA, xsA, xdA, dsA, outA,
                 srcB, dstB, xsB, xdB, dsB, outB,
                 par_v, x_sp, sz_sp, isemA, gsemA, ssemA,
                 isemB, gsemB, ssemB):
    c = lax.axis_index("c")
    s = lax.axis_index("s")
    wid = c * NSUB + s

    @pl.when(s == 0)
    def _():
        pltpu.sync_copy(x_hbm, x_sp)

    pltpu.sync_copy(par_hbm, par_v)
    pltpu.sync_copy(zeros_hbm, sz_sp.at[pl.ds(s * NPS, NPS)])
    plsc.subcore_barrier()

    cs = [par_v[h] for h in range(H_K)]
    cd = [par_v[H_K + h] for h in range(H_K)]
    pp = [par_v[2 * H_K + h] for h in range(H_K)]
    bufs = ((srcA, dstA, xsA, xdA, dsA, outA, isemA, gsemA, ssemA),
            (srcB, dstB, xsB, xdB, dsB, outB, isemB, gsemB, ssemB))
    _sweep_loop(src3_hbm, dst3_hbm, bufs, x_sp, sz_sp, cs, cd, pp, wid)
    _writeback(out_hbm, sz_sp, c, s)


def _sweepn_body(szprev_hbm, xprev_hbm, xres_hbm, src3_hbm, dst3_hbm,
                 par_hbm, zeros_hbm,
                 out_hbm, xout_hbm, pb_hbm,
                 srcA, dstA, xsA, xdA, dsA, outA,
                 srcB, dstB, xsB, xdB, dsB, outB,
                 par_v, xp_v, o_v, st_v, x_sp, sz_sp, stats_sp,
                 isemA, gsemA, ssemA, isemB, gsemB, ssemB):
    c = lax.axis_index("c")
    s = lax.axis_index("s")
    wid = c * NSUB + s
    iota = lax.iota(jnp.int32, 16)

    pltpu.sync_copy(par_hbm, par_v)
    pltpu.sync_copy(zeros_hbm, sz_sp.at[pl.ds(s * NPS, NPS)])
    pltpu.sync_copy(xprev_hbm.at[pl.ds(s * NPS, NPS)], xp_v)

    cs = [par_v[h] for h in range(H_K)]
    cd = [par_v[H_K + h] for h in range(H_K)]
    whv = [par_v[2 * H_K + h] for h in range(H_K)]
    ppv = [par_v[3 * H_K + h] for h in range(H_K)]
    bias = par_v[16]
    gamma = par_v[17]
    beta = par_v[18]
    xresb = par_v[19]

    sumv = jnp.zeros((16,), jnp.float32)
    sqv = jnp.zeros((16,), jnp.float32)
    maxv = jnp.full((16,), -3e38, jnp.float32)
    minv = jnp.full((16,), 3e38, jnp.float32)
    r0 = s * NPS
    for k in range(NPS // NBLK):
        pltpu.sync_copy(szprev_hbm.at[0, pl.ds(r0 + k * NBLK, NBLK)],
                        outA.at[pl.ds(0, NBLK)])
        pltpu.sync_copy(szprev_hbm.at[1, pl.ds(r0 + k * NBLK, NBLK)],
                        outB.at[pl.ds(0, NBLK)])

        def blk(g, carry):
            sm, sq, mx, mn = carry
            rowb = g * 16 + iota
            xp = xp_v[pl.ds(k * NBLK + g * 16, 16)]
            acc = bias * 1.0
            for h in range(H_K):
                s_h = (plsc.load_gather(outA, [rowb, jnp.full((16,), h, jnp.int32)])
                       + plsc.load_gather(outB, [rowb, jnp.full((16,), h, jnp.int32)]))
                z_h = (plsc.load_gather(outA, [rowb, jnp.full((16,), H_K + h, jnp.int32)])
                       + plsc.load_gather(outB, [rowb, jnp.full((16,), H_K + h, jnp.int32)]))
                cdxp = cd[h] * xp
                u0 = cs[h] * xp + cdxp
                a0 = jnp.maximum(u0, 0.2 * u0)
                t0 = ppv[h] + cdxp
                b0 = jnp.maximum(t0, 0.2 * t0)
                w0 = jnp.exp(a0 - b0)
                s_h = s_h + w0
                z_h = z_h + xp * w0
                acc = acc + (0.25 * whv[h]) * (z_h / s_h)
            o_v[pl.ds(k * NBLK + g * 16, 16)] = acc
            grow = r0 + k * NBLK + g * 16 + iota
            m = grow < N_NODES_K
            am = jnp.where(m, acc, 0.0)
            sm = sm + am
            sq = sq + am * acc
            mx = jnp.maximum(mx, jnp.where(m, acc, -3e38))
            mn = jnp.minimum(mn, jnp.where(m, acc, 3e38))
            return sm, sq, mx, mn

        sumv, sqv, maxv, minv = lax.fori_loop(0, NBLK // 16, blk,
                                              (sumv, sqv, maxv, minv))

    st_v[0] = sumv
    st_v[1] = sqv
    st_v[2] = maxv
    st_v[3] = minv
    pltpu.sync_copy(st_v, stats_sp.at[pl.ds(s * 4, 4)])
    plsc.subcore_barrier()
    pltpu.sync_copy(stats_sp, par_v.at[pl.ds(20, NSUB * 4)])

    tsum = jnp.zeros((16,), jnp.float32)
    tsq = jnp.zeros((16,), jnp.float32)
    tmax = jnp.full((16,), -3e38, jnp.float32)
    tmin = jnp.full((16,), 3e38, jnp.float32)
    for si in range(NSUB):
        tsum = tsum + par_v[20 + si * 4]
        tsq = tsq + par_v[20 + si * 4 + 1]
        tmax = jnp.maximum(tmax, par_v[20 + si * 4 + 2])
        tmin = jnp.minimum(tmin, par_v[20 + si * 4 + 3])
    ninv = 1.0 / N_NODES_K
    mean = _bcast(jnp.sum(tsum) * ninv)
    var = jnp.maximum(_bcast(jnp.sum(tsq) * ninv) - mean * mean, 0.0)
    omax = _bcast(jnp.max(tmax))
    omin = _bcast(jnp.min(tmin))
    vv = var + 1e-5
    y = plsc.bitcast(0x5F3759DF - lax.shift_right_logical(
        plsc.bitcast(vv, jnp.int32), 1), jnp.float32)
    for _ in range(3):
        y = y * (1.5 - 0.5 * vv * y * y)
    rstd = y
    dev = jnp.maximum(jnp.abs(omax - mean), jnp.abs(omin - mean))
    xbound = dev * rstd * jnp.abs(gamma) + jnp.abs(beta) + xresb
    pp = [jnp.abs(cs[h]) * xbound for h in range(H_K)]

    pltpu.sync_copy(xres_hbm.at[pl.ds(s * NPS, NPS)], xp_v)

    def ph2(g, carry):
        o = o_v[pl.ds(g * 16, 16)]
        hh = (o - mean) * rstd * gamma + beta
        hh = jnp.maximum(hh, 0.01 * hh)
        o_v[pl.ds(g * 16, 16)] = hh + xp_v[pl.ds(g * 16, 16)]
        return carry

    lax.fori_loop(0, NPS // 16, ph2, 0)
    pltpu.sync_copy(o_v, x_sp.at[pl.ds(s * NPS, NPS)])

    @pl.when(c == 0)
    def _():
        pltpu.sync_copy(o_v, xout_hbm.at[pl.ds(s * NPS, NPS)])

    @pl.when(jnp.logical_and(c == 0, s == 0))
    def _():
        for h in range(H_K):
            st_v[h] = pp[h]
        pltpu.sync_copy(st_v, pb_hbm)

    plsc.subcore_barrier()

    bufs = ((srcA, dstA, xsA, xdA, dsA, outA, isemA, gsemA, ssemA),
            (srcB, dstB, xsB, xdB, dsB, outB, isemB, gsemB, ssemB))
    _sweep_loop(src3_hbm, dst3_hbm, bufs, x_sp, sz_sp, cs, cd, pp, wid)
    _writeback(out_hbm, sz_sp, c, s)


_common_scratch = [
    pltpu.VMEM((SUBSC, 128), jnp.int32),
    pltpu.VMEM((SUBSC, 128), jnp.int32),
    pltpu.VMEM((SUBSC, 128), jnp.float32),
    pltpu.VMEM((SUBSC, 128), jnp.float32),
    pltpu.VMEM((SUBSC, 128), jnp.int32),
    pltpu.VMEM((CHUNK, 2 * H_K), jnp.float32),
    pltpu.VMEM((SUBSC, 128), jnp.int32),
    pltpu.VMEM((SUBSC, 128), jnp.int32),
    pltpu.VMEM((SUBSC, 128), jnp.float32),
    pltpu.VMEM((SUBSC, 128), jnp.float32),
    pltpu.VMEM((SUBSC, 128), jnp.int32),
    pltpu.VMEM((CHUNK, 2 * H_K), jnp.float32),
]
_sems = [pltpu.SemaphoreType.DMA] * 6
_mesh = plsc.VectorSubcoreMesh(core_axis_name="c", subcore_axis_name="s",
                               num_cores=NCORES, num_subcores=NSUB)
_cparams = pltpu.CompilerParams(needs_layout_passes=False,
                                use_tc_tiling_on_sc=False)

_sweep0 = pl.kernel(
    _sweep0_body,
    out_type=jax.ShapeDtypeStruct((NCORES, N_PAD, 2 * H_K), jnp.float32),
    mesh=_mesh,
    compiler_params=_cparams,
    scratch_types=_common_scratch + [
        pltpu.VMEM((3 * H_K, 16), jnp.float32),
        pltpu.VMEM_SHARED((N_PAD,), jnp.float32),
        pltpu.VMEM_SHARED((N_PAD, 2 * H_K), jnp.float32),
    ] + _sems,
)

_sweepn = pl.kernel(
    _sweepn_body,
    out_type=(jax.ShapeDtypeStruct((NCORES, N_PAD, 2 * H_K), jnp.float32),
              jax.ShapeDtypeStruct((N_PAD,), jnp.float32),
              jax.ShapeDtypeStruct((4, 16), jnp.float32)),
    mesh=_mesh,
    compiler_params=_cparams,
    scratch_types=_common_scratch + [
        pltpu.VMEM((20 + NSUB * 4, 16), jnp.float32),
        pltpu.VMEM((NPS,), jnp.float32),
        pltpu.VMEM((NPS,), jnp.float32),
        pltpu.VMEM((4, 16), jnp.float32),
        pltpu.VMEM_SHARED((N_PAD,), jnp.float32),
        pltpu.VMEM_SHARED((N_PAD, 2 * H_K), jnp.float32),
        pltpu.VMEM_SHARED((NSUB * 4, 16), jnp.float32),
    ] + _sems,
)


def _lrelu(v, a):
    return jnp.maximum(v, a * v)


def kernel(x, edge_index, batch, W, att_src, att_dst, gat_bias, bn_gamma,
           bn_beta, decision_vec):
    n = x.shape[0]
    num_graphs = n // decision_vec.shape[0]
    xf = x[:, 0]
    src3 = edge_index[0].reshape(NCHUNKS, SUBSC, 128)
    dst3 = edge_index[1].reshape(NCHUNKS, SUBSC, 128)
    npadch = NCH_PAD - NCHUNKS
    pad_src = jnp.zeros((npadch, SUBSC, 128), jnp.int32)
    pad_dst = (N_NODES_K + (jnp.arange(npadch * CHUNK, dtype=jnp.int32)
                            % (N_PAD - N_NODES_K))).reshape(npadch, SUBSC, 128)
    src3p = jnp.concatenate([src3, pad_src])
    dst3p = jnp.concatenate([dst3, pad_dst])
    cs = W[0] * att_src[0, :, 0]
    cd = W[0] * att_dst[0, :, 0]
    wh = W[0]
    zeros = jnp.zeros((NPS, 2 * H_K), jnp.float32)
    x_res = xf
    xpad = jnp.pad(xf, (0, N_PAD - n))

    xmax = xf.max()
    xmin = xf.min()
    pb0 = jnp.where(cs >= 0, cs * xmax, cs * xmin)
    par0 = jnp.tile(jnp.concatenate([cs, cd, pb0])[:, None], (1, 16))
    sz = _sweep0(xpad, src3p, dst3p, par0.astype(jnp.float32), zeros)

    xresb = jnp.abs(xpad).max()
    head_rows = jnp.concatenate([cs, cd, wh])[:, None]
    scal_rows = jnp.stack([gat_bias[0], bn_gamma[0], bn_beta[0], xresb])[:, None]
    parn_base = jnp.concatenate([
        jnp.tile(head_rows, (1, 16)),
        jnp.zeros((4, 16), jnp.float32),
        jnp.tile(scal_rows, (1, 16)),
        jnp.zeros((NSUB * 4, 16), jnp.float32),
    ]).astype(jnp.float32)
    xprev = xpad
    pb = jnp.tile(pb0[:, None], (1, 16)).astype(jnp.float32)
    for _ in range(2):
        parn = parn_base.at[12:16].set(pb)
        sz, xprev, pb = _sweepn(sz, xprev, xpad, src3p, dst3p, parn, zeros)

    xf2 = xprev[:n]
    pbv = pb[:, 0]
    szc = sz[0, :n] + sz[1, :n]
    s_acc = szc[:, :H_K]
    z_acc = szc[:, H_K:]
    u0 = (cs + cd)[None, :] * xf2[:, None]
    a0 = _lrelu(u0, 0.2)
    t0 = pbv[None, :] + cd[None, :] * xf2[:, None]
    b0 = _lrelu(t0, 0.2)
    w0 = jnp.exp(a0 - b0)
    s_acc = s_acc + w0
    z_acc = z_acc + xf2[:, None] * w0
    out = gat_bias[0] + (wh[None, :] * z_acc / s_acc).mean(axis=1)
    mean = out.mean()
    var = ((out - mean) ** 2).mean()
    h = (out - mean) / jnp.sqrt(var + 1e-5) * bn_gamma[0] + bn_beta[0]
    h = _lrelu(h, 0.01)
    xf3 = h + x_res

    xm = xf3.reshape(num_graphs, -1) * decision_vec[None, :]
    pooled = xm.mean(axis=1)[:, None]
    return (pooled - pooled.min()) / (pooled.max() - pooled.min() + 1e-6)


# speedup vs baseline: 1.3461x; 1.3461x over previous
import jax
import jax.numpy as jnp
from jax import lax
from jax.experimental import pallas as pl
from jax.experimental.pallas import tpu as pltpu
from jax.experimental.pallas import tpu_sc as plsc

N_NODES_K = 100000
N_EDGES_K = 6400000
H_K = 4
CHUNK = 2048
SUBSC = CHUNK // 128
NCORES = 2
NSUB = 16
NW = NCORES * NSUB
NCH_PAD = 3136
NCHUNKS = N_EDGES_K // CHUNK
CPW = NCH_PAD // NW
NPS = 6272
N_PAD = NPS * NSUB
NBLK = 1568


def _bcast(v):
    return jnp.full((16,), v, jnp.float32)


def _vlrelu(v, a):
    return jnp.maximum(v, a * v)


def _sweep_loop(src3_hbm, dst3_hbm, bufs, x_sp, sz_sp, cs, cd, pp, wid):
    iota = lax.iota(jnp.int32, 16)
    lo = wid * CPW

    def fire_idx(ch, p):
        src2, dst2, isem = bufs[p][0], bufs[p][1], bufs[p][6]
        chc = jnp.minimum(ch, NCH_PAD - 1)
        pltpu.async_copy(src3_hbm.at[chc], src2, isem)
        pltpu.async_copy(dst3_hbm.at[chc], dst2, isem)

    def wait_idx(p):
        src2, dst2, isem = bufs[p][0], bufs[p][1], bufs[p][6]
        pltpu.make_async_copy(src3_hbm.at[0], src2, isem).wait()
        pltpu.make_async_copy(dst3_hbm.at[0], dst2, isem).wait()

    def fire_gathers(p):
        src2, dst2, xs_v, xd_v, gsem = (bufs[p][0], bufs[p][1], bufs[p][2],
                                        bufs[p][3], bufs[p][7])
        for j in range(SUBSC):
            pltpu.async_copy(x_sp.at[src2.at[j]], xs_v.at[j], gsem)
            pltpu.async_copy(x_sp.at[dst2.at[j]], xd_v.at[j], gsem)

    def wait_gathers(p):
        src2, dst2, xs_v, xd_v, gsem = (bufs[p][0], bufs[p][1], bufs[p][2],
                                        bufs[p][3], bufs[p][7])
        for j in range(SUBSC):
            pltpu.make_async_copy(x_sp.at[src2.at[j]], xs_v.at[j], gsem).wait()
            pltpu.make_async_copy(x_sp.at[dst2.at[j]], xd_v.at[j], gsem).wait()

    def fire_scatters(p):
        ds_v, out_v, ssem = bufs[p][4], bufs[p][5], bufs[p][8]
        for j in range(SUBSC):
            pltpu.async_copy(out_v.at[pl.ds(j * 128, 128)],
                             sz_sp.at[ds_v.at[j]], ssem, add=True)

    def drain_scatters(p):
        ds_v, out_v, ssem = bufs[p][4], bufs[p][5], bufs[p][8]
        for j in range(SUBSC):
            pltpu.make_async_copy(out_v.at[pl.ds(j * 128, 128)],
                                  sz_sp.at[ds_v.at[j]], ssem).wait()

    def compute(p):
        dst2, xs_v, xd_v, ds_v, out_v = (bufs[p][1], bufs[p][2], bufs[p][3],
                                         bufs[p][4], bufs[p][5])
        for j in range(SUBSC):
            def grp(l, carry):
                g = j * 8 + l
                xs = xs_v[j, pl.ds(l * 16, 16)]
                xd = xd_v[j, pl.ds(l * 16, 16)]
                ds_v[j, pl.ds(l * 16, 16)] = dst2[j, pl.ds(l * 16, 16)]
                row = g * 16 + iota
                for h in range(H_K):
                    cdxd = cd[h] * xd
                    u = cs[h] * xs + cdxd
                    a = jnp.maximum(u, 0.2 * u)
                    t = pp[h] + cdxd
                    b = jnp.maximum(t, 0.2 * t)
                    w = jnp.exp(a - b)
                    plsc.store_scatter(out_v,
                                       [row, jnp.full((16,), h, jnp.int32)], w)
                    plsc.store_scatter(out_v,
                                       [row, jnp.full((16,), H_K + h, jnp.int32)],
                                       xs * w)
                return carry

            lax.fori_loop(0, 8, grp, 0)

    fire_idx(lo, 0)
    fire_idx(lo + 1, 1)

    def pair_body(it, carry):
        c0 = lo + 2 * it
        for p in range(2):
            wait_idx(p)
            fire_gathers(p)
        for p in range(2):
            @pl.when(it > 0)
            def _():
                drain_scatters(p)

            wait_gathers(p)
            compute(p)
            fire_idx(c0 + p + 2, p)
            fire_scatters(p)
        return carry

    lax.fori_loop(0, CPW // 2, pair_body, 0)
    for p in range(2):
        drain_scatters(p)
        wait_idx(p)


def _writeback(out_hbm, sz_sp, c, s):
    plsc.subcore_barrier()
    pltpu.sync_copy(sz_sp.at[pl.ds(s * NPS, NPS)],
                    out_hbm.at[c, pl.ds(s * NPS, NPS)])


def _sweep0_body(x_hbm, src3_hbm, dst3_hbm, par_hbm, zeros_hbm, out_hbm,
                 srcA, dstA, xsA, xdA, dsA, outA,
                 srcB, dstB, xsB, xdB, dsB, outB,
                 par_v, x_sp, sz_sp, isemA, gsemA, ssemA,
                 isemB, gsemB, ssemB):
    c = lax.axis_index("c")
    s = lax.axis_index("s")
    wid = c * NSUB + s

    @pl.when(s == 0)
    def _():
        pltpu.sync_copy(x_hbm, x_sp)

    pltpu.sync_copy(par_hbm, par_v)
    pltpu.sync_copy(zeros_hbm, sz_sp.at[pl.ds(s * NPS, NPS)])
    plsc.subcore_barrier()

    cs = [par_v[h] for h in range(H_K)]
    cd = [par_v[H_K + h] for h in range(H_K)]
    pp = [par_v[2 * H_K + h] for h in range(H_K)]
    bufs = ((srcA, dstA, xsA, xdA, dsA, outA, isemA, gsemA, ssemA),
            (srcB, dstB, xsB, xdB, dsB, outB, isemB, gsemB, ssemB))
    _sweep_loop(src3_hbm, dst3_hbm, bufs, x_sp, sz_sp, cs, cd, pp, wid)
    _writeback(out_hbm, sz_sp, c, s)


def _sweepn_body(szprev_hbm, xprev_hbm, xres_hbm, src3_hbm, dst3_hbm,
                 par_hbm, zeros_hbm,
                 out_hbm, xout_hbm, pb_hbm,
                 srcA, dstA, xsA, xdA, dsA, outA,
                 srcB, dstB, xsB, xdB, dsB, outB,
                 par_v, xp_v, o_v, st_v, x_sp, sz_sp, stats_sp,
                 isemA, gsemA, ssemA, isemB, gsemB, ssemB):
    c = lax.axis_index("c")
    s = lax.axis_index("s")
    wid = c * NSUB + s
    iota = lax.iota(jnp.int32, 16)

    pltpu.sync_copy(par_hbm, par_v)
    pltpu.sync_copy(zeros_hbm, sz_sp.at[pl.ds(s * NPS, NPS)])
    pltpu.sync_copy(xprev_hbm.at[pl.ds(s * NPS, NPS)], xp_v)

    cs = [par_v[h] for h in range(H_K)]
    cd = [par_v[H_K + h] for h in range(H_K)]
    whv = [par_v[2 * H_K + h] for h in range(H_K)]
    ppv = [par_v[3 * H_K + h] for h in range(H_K)]
    bias = par_v[16]
    gamma = par_v[17]
    beta = par_v[18]
    xresb = par_v[19]

    sumv = jnp.zeros((16,), jnp.float32)
    sqv = jnp.zeros((16,), jnp.float32)
    maxv = jnp.full((16,), -3e38, jnp.float32)
    minv = jnp.full((16,), 3e38, jnp.float32)
    r0 = s * NPS
    for k in range(NPS // NBLK):
        pltpu.sync_copy(szprev_hbm.at[0, pl.ds(r0 + k * NBLK, NBLK)],
                        outA.at[pl.ds(0, NBLK)])
        pltpu.sync_copy(szprev_hbm.at[1, pl.ds(r0 + k * NBLK, NBLK)],
                        outB.at[pl.ds(0, NBLK)])

        def blk(g, carry):
            sm, sq, mx, mn = carry
            rowb = g * 16 + iota
            xp = xp_v[pl.ds(k * NBLK + g * 16, 16)]
            acc = bias * 1.0
            for h in range(H_K):
                s_h = (plsc.load_gather(outA, [rowb, jnp.full((16,), h, jnp.int32)])
                       + plsc.load_gather(outB, [rowb, jnp.full((16,), h, jnp.int32)]))
                z_h = (plsc.load_gather(outA, [rowb, jnp.full((16,), H_K + h, jnp.int32)])
                       + plsc.load_gather(outB, [rowb, jnp.full((16,), H_K + h, jnp.int32)]))
                cdxp = cd[h] * xp
                u0 = cs[h] * xp + cdxp
                a0 = jnp.maximum(u0, 0.2 * u0)
                t0 = ppv[h] + cdxp
                b0 = jnp.maximum(t0, 0.2 * t0)
                w0 = jnp.exp(a0 - b0)
                s_h = s_h + w0
                z_h = z_h + xp * w0
                acc = acc + (0.25 * whv[h]) * (z_h / s_h)
            o_v[pl.ds(k * NBLK + g * 16, 16)] = acc
            grow = r0 + k * NBLK + g * 16 + iota
            m = grow < N_NODES_K
            am = jnp.where(m, acc, 0.0)
            sm = sm + am
            sq = sq + am * acc
            mx = jnp.maximum(mx, jnp.where(m, acc, -3e38))
            mn = jnp.minimum(mn, jnp.where(m, acc, 3e38))
            return sm, sq, mx, mn

        sumv, sqv, maxv, minv = lax.fori_loop(0, NBLK // 16, blk,
                                              (sumv, sqv, maxv, minv))

    st_v[0] = sumv
    st_v[1] = sqv
    st_v[2] = maxv
    st_v[3] = minv
    pltpu.sync_copy(st_v, stats_sp.at[pl.ds(s * 4, 4)])
    plsc.subcore_barrier()
    pltpu.sync_copy(stats_sp, par_v.at[pl.ds(20, NSUB * 4)])

    tsum = jnp.zeros((16,), jnp.float32)
    tsq = jnp.zeros((16,), jnp.float32)
    tmax = jnp.full((16,), -3e38, jnp.float32)
    tmin = jnp.full((16,), 3e38, jnp.float32)
    for si in range(NSUB):
        tsum = tsum + par_v[20 + si * 4]
        tsq = tsq + par_v[20 + si * 4 + 1]
        tmax = jnp.maximum(tmax, par_v[20 + si * 4 + 2])
        tmin = jnp.minimum(tmin, par_v[20 + si * 4 + 3])
    ninv = 1.0 / N_NODES_K
    mean = _bcast(jnp.sum(tsum) * ninv)
    var = jnp.maximum(_bcast(jnp.sum(tsq) * ninv) - mean * mean, 0.0)
    omax = _bcast(jnp.max(tmax))
    omin = _bcast(jnp.min(tmin))
    vv = var + 1e-5
    y = plsc.bitcast(0x5F3759DF - lax.shift_right_logical(
        plsc.bitcast(vv, jnp.int32), 1), jnp.float32)
    for _ in range(3):
        y = y * (1.5 - 0.5 * vv * y * y)
    rstd = y
    dev = jnp.maximum(jnp.abs(omax - mean), jnp.abs(omin - mean))
    xbound = dev * rstd * jnp.abs(gamma) + jnp.abs(beta) + xresb
    pp = [jnp.abs(cs[h]) * xbound for h in range(H_K)]

    pltpu.sync_copy(xres_hbm.at[pl.ds(s * NPS, NPS)], xp_v)

    def ph2(g, carry):
        o = o_v[pl.ds(g * 16, 16)]
        hh = (o - mean) * rstd * gamma + beta
        hh = jnp.maximum(hh, 0.01 * hh)
        o_v[pl.ds(g * 16, 16)] = hh + xp_v[pl.ds(g * 16, 16)]
        return carry

    lax.fori_loop(0, NPS // 16, ph2, 0)
    pltpu.sync_copy(o_v, x_sp.at[pl.ds(s * NPS, NPS)])

    @pl.when(c == 0)
    def _():
        pltpu.sync_copy(o_v, xout_hbm.at[pl.ds(s * NPS, NPS)])

    @pl.when(jnp.logical_and(c == 0, s == 0))
    def _():
        for h in range(H_K):
            st_v[h] = pp[h]
        pltpu.sync_copy(st_v, pb_hbm)

    plsc.subcore_barrier()

    bufs = ((srcA, dstA, xsA, xdA, dsA, outA, isemA, gsemA, ssemA),
            (srcB, dstB, xsB, xdB, dsB, outB, isemB, gsemB, ssemB))
    _sweep_loop(src3_hbm, dst3_hbm, bufs, x_sp, sz_sp, cs, cd, pp, wid)
    _writeback(out_hbm, sz_sp, c, s)


_common_scratch = [
    pltpu.VMEM((SUBSC, 128), jnp.int32),
    pltpu.VMEM((SUBSC, 128), jnp.int32),
    pltpu.VMEM((SUBSC, 128), jnp.float32),
    pltpu.VMEM((SUBSC, 128), jnp.float32),
    pltpu.VMEM((SUBSC, 128), jnp.int32),
    pltpu.VMEM((CHUNK, 2 * H_K), jnp.float32),
    pltpu.VMEM((SUBSC, 128), jnp.int32),
    pltpu.VMEM((SUBSC, 128), jnp.int32),
    pltpu.VMEM((SUBSC, 128), jnp.float32),
    pltpu.VMEM((SUBSC, 128), jnp.float32),
    pltpu.VMEM((SUBSC, 128), jnp.int32),
    pltpu.VMEM((CHUNK, 2 * H_K), jnp.float32),
]
_sems = [pltpu.SemaphoreType.DMA] * 6
_mesh = plsc.VectorSubcoreMesh(core_axis_name="c", subcore_axis_name="s",
                               num_cores=NCORES, num_subcores=NSUB)
_cparams = pltpu.CompilerParams(needs_layout_passes=False,
                                use_tc_tiling_on_sc=False)

_sweep0 = pl.kernel(
    _sweep0_body,
    out_type=jax.ShapeDtypeStruct((NCORES, N_PAD, 2 * H_K), jnp.float32),
    mesh=_mesh,
    compiler_params=_cparams,
    scratch_types=_common_scratch + [
        pltpu.VMEM((3 * H_K, 16), jnp.float32),
        pltpu.VMEM_SHARED((N_PAD,), jnp.float32),
        pltpu.VMEM_SHARED((N_PAD, 2 * H_K), jnp.float32),
    ] + _sems,
)

_sweepn = pl.kernel(
    _sweepn_body,
    out_type=(jax.ShapeDtypeStruct((NCORES, N_PAD, 2 * H_K), jnp.float32),
              jax.ShapeDtypeStruct((N_PAD,), jnp.float32),
              jax.ShapeDtypeStruct((4, 16), jnp.float32)),
    mesh=_mesh,
    compiler_params=_cparams,
    scratch_types=_common_scratch + [
        pltpu.VMEM((20 + NSUB * 4, 16), jnp.float32),
        pltpu.VMEM((NPS,), jnp.float32),
        pltpu.VMEM((NPS,), jnp.float32),
        pltpu.VMEM((4, 16), jnp.float32),
        pltpu.VMEM_SHARED((N_PAD,), jnp.float32),
        pltpu.VMEM_SHARED((N_PAD, 2 * H_K), jnp.float32),
        pltpu.VMEM_SHARED((NSUB * 4, 16), jnp.float32),
    ] + _sems,
)


def _lrelu(v, a):
    return jnp.maximum(v, a * v)


def kernel(x, edge_index, batch, W, att_src, att_dst, gat_bias, bn_gamma,
           bn_beta, decision_vec):
    n = x.shape[0]
    num_graphs = n // decision_vec.shape[0]
    xf = x[:, 0]
    src3 = edge_index[0].reshape(NCHUNKS, SUBSC, 128)
    dst3 = edge_index[1].reshape(NCHUNKS, SUBSC, 128)
    npadch = NCH_PAD - NCHUNKS
    pad_src = jnp.zeros((npadch, SUBSC, 128), jnp.int32)
    pad_dst = (N_NODES_K + (jnp.arange(npadch * CHUNK, dtype=jnp.int32)
                            % (N_PAD - N_NODES_K))).reshape(npadch, SUBSC, 128)
    src3p = jnp.concatenate([src3, pad_src])
    dst3p = jnp.concatenate([dst3, pad_dst])
    cs = W[0] * att_src[0, :, 0]
    cd = W[0] * att_dst[0, :, 0]
    wh = W[0]
    zeros = jnp.zeros((NPS, 2 * H_K), jnp.float32)
    x_res = xf
    xpad = jnp.pad(xf, (0, N_PAD - n))

    xmax = xf.max()
    xmin = xf.min()
    pb0 = jnp.where(cs >= 0, cs * xmax, cs * xmin)
    par0 = jnp.tile(jnp.concatenate([cs, cd, pb0])[:, None], (1, 16))
    sz = _sweep0(xpad, src3p, dst3p, par0.astype(jnp.float32), zeros)

    xresb = jnp.abs(xpad).max()
    head_rows = jnp.concatenate([cs, cd, wh])[:, None]
    scal_rows = jnp.stack([gat_bias[0], bn_gamma[0], bn_beta[0], xresb])[:, None]
    parn_base = jnp.concatenate([
        jnp.tile(head_rows, (1, 16)),
        jnp.zeros((4, 16), jnp.float32),
        jnp.tile(scal_rows, (1, 16)),
        jnp.zeros((NSUB * 4, 16), jnp.float32),
    ]).astype(jnp.float32)
    xprev = xpad
    pb = jnp.tile(pb0[:, None], (1, 16)).astype(jnp.float32)
    for _ in range(2):
        parn = parn_base.at[12:16].set(pb)
        sz, xprev, pb = _sweepn(sz, xprev, xpad, src3p, dst3p, parn, zeros)

    xf2 = xprev[:n]
    pbv = pb[:, 0]
    szc = sz[0, :n] + sz[1, :n]
    s_acc = szc[:, :H_K]
    z_acc = szc[:, H_K:]
    u0 = (cs + cd)[None, :] * xf2[:, None]
    a0 = _lrelu(u0, 0.2)
    t0 = pbv[None, :] + cd[None, :] * xf2[:, None]
    b0 = _lrelu(t0, 0.2)
    w0 = jnp.exp(a0 - b0)
    s_acc = s_acc + w0
    z_acc = z_acc + xf2[:, None] * w0
    out = gat_bias[0] + (wh[None, :] * z_acc / s_acc).mean(axis=1)
    mean = out.mean()
    var = ((out - mean) ** 2).mean()
    h = (out - mean) / jnp.sqrt(var + 1e-5) * bn_gamma[0] + bn_beta[0]
    h = _lrelu(h, 0.01)
    xf3 = h + x_res

    xm = xf3.reshape(num_graphs, -1) * decision_vec[None, :]
    pooled = xm.mean(axis=1)[:, None]
    return (pooled - pooled.min()) / (pooled.max() - pooled.min() + 1e-6)
